# Initial kernel scaffold; baseline (speedup 1.0000x reference)
import jax
import jax.numpy as jnp
from jax.experimental import pallas as pl


def kernel(x, edge_index, batch, W1, b1, W2, b2, Wl1, bl1, Wl2, bl2):
    raise NotImplementedError("write your pallas kernel here")


# speedup vs baseline: 21.9200x; 21.9200x over previous
import functools

import jax
import jax.numpy as jnp
from jax import lax
from jax.experimental import pallas as pl
from jax.experimental.pallas import tpu as pltpu
from jax.experimental.pallas import tpu_sc as plsc

N = 10000
E = 320000
D = 128
NG = 16
NC = 2
NS = 16
NW = NC * NS
EPT = E // NW
K = 125
CHUNKS = EPT // K
WT = 10
RPT = N // WT
ZR = 40
R = 1000
G = N // R

_f32 = jnp.float32
_HI = lax.Precision.HIGHEST


def _zero_fill(ref, nrows, ncols):
    z = jnp.zeros((16,), _f32)

    def body(r, _):
        for c in range(ncols // 16):
            ref[r, pl.ds(c * 16, 16)] = z
        return 0

    lax.fori_loop(0, nrows, body, 0)


def _hist_body(dst_hbm, out_hbm, dstb, ones_v, zb, deg_sh):
    cid = lax.axis_index("c")
    sid = lax.axis_index("s")
    wid = cid * NS + sid
    pltpu.sync_copy(dst_hbm.at[pl.ds(wid * CHUNKS, CHUNKS)], dstb)
    one = jnp.ones((16,), _f32)

    def fill(r, _):
        ones_v[r, :] = one
        return 0

    lax.fori_loop(0, K, fill, 0)
    _zero_fill(zb, ZR, 16)

    @pl.when(sid < WT)
    def _zero():
        for k in range(RPT // ZR):
            pltpu.sync_copy(zb, deg_sh.at[pl.ds(sid * RPT + k * ZR, ZR)])

    plsc.subcore_barrier()

    def chunk(j, _):
        pltpu.sync_copy(ones_v, deg_sh.at[dstb.at[j]], add=True)
        return 0

    lax.fori_loop(0, CHUNKS, chunk, 0)
    plsc.subcore_barrier()

    @pl.when(sid < WT)
    def _wb():
        pltpu.sync_copy(deg_sh.at[pl.ds(sid * RPT, RPT)],
                        out_hbm.at[pl.ds(cid * N + sid * RPT, RPT)])


@functools.cache
def _hist_kernel():
    mesh = plsc.VectorSubcoreMesh(core_axis_name="c", subcore_axis_name="s")
    return pl.kernel(
        _hist_body,
        out_type=jax.ShapeDtypeStruct((NC * N, 16), _f32),
        mesh=mesh,
        scratch_types=[
            pltpu.VMEM((CHUNKS, K), jnp.int32),
            pltpu.VMEM((K, 16), _f32),
            pltpu.VMEM((ZR, 16), _f32),
            pltpu.VMEM_SHARED((N, 16), _f32),
        ],
    )


def _scatter_body(hs_hbm, src_hbm, dst_hbm, out_hbm, srcb, dstb, rows, zb,
                  acc_sh, sem):
    cid = lax.axis_index("c")
    sid = lax.axis_index("s")
    wid = cid * NS + sid
    pltpu.sync_copy(src_hbm.at[pl.ds(wid * CHUNKS, CHUNKS)], srcb)
    pltpu.sync_copy(dst_hbm.at[pl.ds(wid * CHUNKS, CHUNKS)], dstb)
    _zero_fill(zb, ZR, D)

    @pl.when(sid < WT)
    def _zero():
        for k in range(RPT // ZR):
            pltpu.sync_copy(zb, acc_sh.at[pl.ds(sid * RPT + k * ZR, ZR)])

    plsc.subcore_barrier()

    def chunk(j, _):
        pltpu.async_copy(hs_hbm.at[srcb.at[j]], rows, sem).wait()
        pltpu.sync_copy(rows, acc_sh.at[dstb.at[j]], add=True)
        return 0

    lax.fori_loop(0, CHUNKS, chunk, 0)
    plsc.subcore_barrier()

    @pl.when(sid < WT)
    def _wb():
        pltpu.sync_copy(acc_sh.at[pl.ds(sid * RPT, RPT)],
                        out_hbm.at[pl.ds(cid * N + sid * RPT, RPT)])


@functools.cache
def _scatter_kernel():
    mesh = plsc.VectorSubcoreMesh(core_axis_name="c", subcore_axis_name="s")
    return pl.kernel(
        _scatter_body,
        out_type=jax.ShapeDtypeStruct((NC * N, D), _f32),
        mesh=mesh,
        scratch_types=[
            pltpu.VMEM((CHUNKS, K), jnp.int32),
            pltpu.VMEM((CHUNKS, K), jnp.int32),
            pltpu.VMEM((K, D), _f32),
            pltpu.VMEM((ZR, D), _f32),
            pltpu.VMEM_SHARED((N, D), _f32),
            pltpu.SemaphoreType.DMA,
        ],
    )


def _dis_of(hist_ref):
    deg = 1.0 + hist_ref[0, :, 0] + hist_ref[1, :, 0]
    return lax.rsqrt(deg)


def _b1_body(x_ref, w_ref, hist_ref, out_ref):
    dis = _dis_of(hist_ref)
    hw = jnp.dot(x_ref[...], w_ref[...], preferred_element_type=_f32,
                 precision=_HI)
    out_ref[...] = hw * dis[:, None]


def _b2_body(acc_ref, hs_ref, b_ref, w_ref, hist_ref, out_ref):
    dis = _dis_of(hist_ref)
    h = dis[:, None] * (acc_ref[0] + acc_ref[1] + hs_ref[...]) + b_ref[...]
    h = jnp.maximum(h, 0.0)
    hw = jnp.dot(h, w_ref[...], preferred_element_type=_f32, precision=_HI)
    out_ref[...] = hw * dis[:, None]


def _b3_body(acc_ref, hs_ref, b_ref, hist_ref, batch_ref, gsum_ref, cnt_ref):
    i = pl.program_id(0)
    dis = _dis_of(hist_ref)
    h = dis[:, None] * (acc_ref[0] + acc_ref[1] + hs_ref[...]) + b_ref[...]
    h = jnp.maximum(h, 0.0)
    bt = batch_ref[0, 0, :]
    seg = lax.broadcasted_iota(jnp.int32, (NG, R), 0)
    onehot = (bt[None, :] == seg).astype(_f32)
    part = jnp.dot(onehot, h, preferred_element_type=_f32, precision=_HI)
    cpart = jnp.sum(onehot, axis=1, keepdims=True)

    @pl.when(i == 0)
    def _init():
        gsum_ref[...] = jnp.zeros_like(gsum_ref)
        cnt_ref[...] = jnp.zeros_like(cnt_ref)

    gsum_ref[...] += part
    cnt_ref[...] += cpart


def _b4_body(gsum_ref, cnt_ref, wl1_ref, bl1_ref, wl2_ref, bl2_ref, out_ref):
    g = gsum_ref[...] / jnp.maximum(cnt_ref[...], 1.0)
    a = jnp.dot(g, wl1_ref[...], preferred_element_type=_f32, precision=_HI)
    a = jnp.maximum(a + bl1_ref[...], 0.0)
    o = jnp.dot(a, wl2_ref[...], preferred_element_type=_f32, precision=_HI)
    o = o + bl2_ref[...]
    out_ref[...] = 1.0 / (1.0 + jnp.exp(-o))


def _b1_call(x, W1, hist):
    return pl.pallas_call(
        _b1_body,
        grid=(G,),
        in_specs=[
            pl.BlockSpec((R, D), lambda i: (i, 0)),
            pl.BlockSpec((D, D), lambda i: (0, 0)),
            pl.BlockSpec((NC, R, 16), lambda i: (0, i, 0)),
        ],
        out_specs=pl.BlockSpec((R, D), lambda i: (i, 0)),
        out_shape=jax.ShapeDtypeStruct((N, D), _f32),
    )(x, W1, hist)


def _b2_call(acc, hs, b, W, hist):
    return pl.pallas_call(
        _b2_body,
        grid=(G,),
        in_specs=[
            pl.BlockSpec((NC, R, D), lambda i: (0, i, 0)),
            pl.BlockSpec((R, D), lambda i: (i, 0)),
            pl.BlockSpec((1, D), lambda i: (0, 0)),
            pl.BlockSpec((D, D), lambda i: (0, 0)),
            pl.BlockSpec((NC, R, 16), lambda i: (0, i, 0)),
        ],
        out_specs=pl.BlockSpec((R, D), lambda i: (i, 0)),
        out_shape=jax.ShapeDtypeStruct((N, D), _f32),
    )(acc, hs, b, W, hist)


def _b3_call(acc, hs, b, hist, batch3):
    return pl.pallas_call(
        _b3_body,
        grid=(G,),
        in_specs=[
            pl.BlockSpec((NC, R, D), lambda i: (0, i, 0)),
            pl.BlockSpec((R, D), lambda i: (i, 0)),
            pl.BlockSpec((1, D), lambda i: (0, 0)),
            pl.BlockSpec((NC, R, 16), lambda i: (0, i, 0)),
            pl.BlockSpec((1, 1, R), lambda i: (i, 0, 0)),
        ],
        out_specs=[
            pl.BlockSpec((NG, D), lambda i: (0, 0)),
            pl.BlockSpec((NG, 1), lambda i: (0, 0)),
        ],
        out_shape=[
            jax.ShapeDtypeStruct((NG, D), _f32),
            jax.ShapeDtypeStruct((NG, 1), _f32),
        ],
    )(acc, hs, b, hist, batch3)


def _b4_call(gsum, cnt, Wl1, bl1, Wl2, bl2):
    return pl.pallas_call(
        _b4_body,
        out_shape=jax.ShapeDtypeStruct((NG, 5), _f32),
    )(gsum, cnt, Wl1, bl1, Wl2, bl2)


def kernel(x, edge_index, batch, W1, b1, W2, b2, Wl1, bl1, Wl2, bl2):
    src2 = edge_index[0].reshape(NW * CHUNKS, K)
    dst2 = edge_index[1].reshape(NW * CHUNKS, K)
    hist = _hist_kernel()(dst2).reshape(NC, N, 16)
    hs1 = _b1_call(x, W1, hist)
    acc1 = _scatter_kernel()(hs1, src2, dst2).reshape(NC, N, D)
    hs2 = _b2_call(acc1, hs1, b1.reshape(1, D), W2, hist)
    acc2 = _scatter_kernel()(hs2, src2, dst2).reshape(NC, N, D)
    gsum, cnt = _b3_call(acc2, hs2, b2.reshape(1, D), hist,
                         batch.reshape(G, 1, R))
    return _b4_call(gsum, cnt, Wl1, bl1.reshape(1, D // 2), Wl2,
                    bl2.reshape(1, 5))


# speedup vs baseline: 29.5757x; 1.3493x over previous
import functools

import jax
import jax.numpy as jnp
from jax import lax
from jax.experimental import pallas as pl
from jax.experimental.pallas import tpu as pltpu
from jax.experimental.pallas import tpu_sc as plsc

N = 10000
E = 320000
D = 128
NG = 16
NC = 2
NS = 16
NW = NC * NS
EPT = E // NW
K = 100
CHUNKS = EPT // K
HALVES = 2
HALF = CHUNKS // HALVES
WT = 10
RPT = N // WT
ZR = 40
R = 1000
G = N // R

_f32 = jnp.float32
_HI = lax.Precision.HIGHEST


def _hist_body(dst_hbm, out_hbm, dstb, ones_v, zb, deg_sh):
    cid = lax.axis_index("c")
    sid = lax.axis_index("s")
    wid = cid * NS + sid
    one = jnp.ones((16,), _f32)
    z = jnp.zeros((16,), _f32)

    def fill(r, _):
        ones_v[r, :] = one
        zb[r % ZR, :] = z
        return 0

    lax.fori_loop(0, K, fill, 0)

    @pl.when(sid < WT)
    def _zero():
        for k in range(RPT // ZR):
            pltpu.sync_copy(zb, deg_sh.at[pl.ds(sid * RPT + k * ZR, ZR)])

    plsc.subcore_barrier()

    def chunk(j, _):
        pltpu.sync_copy(ones_v, deg_sh.at[dstb.at[j]], add=True)
        return 0

    for h in range(HALVES):
        pltpu.sync_copy(dst_hbm.at[wid, h], dstb)
        lax.fori_loop(0, HALF, chunk, 0)

    plsc.subcore_barrier()

    @pl.when(sid < WT)
    def _wb():
        pltpu.sync_copy(deg_sh.at[pl.ds(sid * RPT, RPT)],
                        out_hbm.at[pl.ds(cid * N + sid * RPT, RPT)])


@functools.cache
def _hist_kernel():
    mesh = plsc.VectorSubcoreMesh(core_axis_name="c", subcore_axis_name="s")
    return pl.kernel(
        _hist_body,
        out_type=jax.ShapeDtypeStruct((NC * N, 16), _f32),
        mesh=mesh,
        scratch_types=[
            pltpu.VMEM((HALF, K), jnp.int32),
            pltpu.VMEM((K, 16), _f32),
            pltpu.VMEM((ZR, 16), _f32),
            pltpu.VMEM_SHARED((N, 16), _f32),
        ],
    )


def _scatter_body(hs_hbm, src_hbm, dst_hbm, zrows_hbm, out_hbm,
                  srcb, dstb, rows0, rows1, acc_sh, sem0, sem1):
    cid = lax.axis_index("c")
    sid = lax.axis_index("s")
    wid = cid * NS + sid

    @pl.when(sid < WT)
    def _zero():
        pltpu.sync_copy(zrows_hbm, acc_sh.at[pl.ds(sid * RPT, RPT)])

    plsc.subcore_barrier()

    def pair(t, _):
        j = 2 * t
        pltpu.async_copy(hs_hbm.at[srcb.at[j + 1]], rows1, sem1)
        pltpu.make_async_copy(hs_hbm.at[srcb.at[j]], rows0, sem0).wait()
        pltpu.sync_copy(rows0, acc_sh.at[dstb.at[j]], add=True)

        @pl.when(j + 2 < HALF)
        def _next():
            pltpu.async_copy(hs_hbm.at[srcb.at[j + 2]], rows0, sem0)

        pltpu.make_async_copy(hs_hbm.at[srcb.at[j + 1]], rows1, sem1).wait()
        pltpu.sync_copy(rows1, acc_sh.at[dstb.at[j + 1]], add=True)
        return 0

    for h in range(HALVES):
        pltpu.sync_copy(src_hbm.at[wid, h], srcb)
        pltpu.sync_copy(dst_hbm.at[wid, h], dstb)
        pltpu.async_copy(hs_hbm.at[srcb.at[0]], rows0, sem0)
        lax.fori_loop(0, HALF // 2, pair, 0)

    plsc.subcore_barrier()

    @pl.when(sid < WT)
    def _wb():
        pltpu.sync_copy(acc_sh.at[pl.ds(sid * RPT, RPT)],
                        out_hbm.at[pl.ds(cid * N + sid * RPT, RPT)])


@functools.cache
def _scatter_kernel():
    mesh = plsc.VectorSubcoreMesh(core_axis_name="c", subcore_axis_name="s")
    return pl.kernel(
        _scatter_body,
        out_type=jax.ShapeDtypeStruct((NC * N, D), _f32),
        mesh=mesh,
        scratch_types=[
            pltpu.VMEM((HALF, K), jnp.int32),
            pltpu.VMEM((HALF, K), jnp.int32),
            pltpu.VMEM((K, D), _f32),
            pltpu.VMEM((K, D), _f32),
            pltpu.VMEM_SHARED((N, D), _f32),
            pltpu.SemaphoreType.DMA,
            pltpu.SemaphoreType.DMA,
        ],
    )


def _dis_of(hist_ref):
    deg = 1.0 + hist_ref[0, :, 0] + hist_ref[1, :, 0]
    return lax.rsqrt(deg)


def _b1_body(x_ref, w_ref, hist_ref, out_ref):
    dis = _dis_of(hist_ref)
    hw = jnp.dot(x_ref[...], w_ref[...], preferred_element_type=_f32,
                 precision=_HI)
    out_ref[...] = hw * dis[:, None]


def _b2_body(acc_ref, hs_ref, b_ref, w_ref, hist_ref, out_ref):
    dis = _dis_of(hist_ref)
    h = dis[:, None] * (acc_ref[0] + acc_ref[1] + hs_ref[...]) + b_ref[...]
    h = jnp.maximum(h, 0.0)
    hw = jnp.dot(h, w_ref[...], preferred_element_type=_f32, precision=_HI)
    out_ref[...] = hw * dis[:, None]


def _b3_body(acc_ref, hs_ref, b_ref, hist_ref, batch_ref, gsum_ref, cnt_ref):
    i = pl.program_id(0)
    dis = _dis_of(hist_ref)
    h = dis[:, None] * (acc_ref[0] + acc_ref[1] + hs_ref[...]) + b_ref[...]
    h = jnp.maximum(h, 0.0)
    bt = batch_ref[0, 0, :]
    seg = lax.broadcasted_iota(jnp.int32, (NG, R), 0)
    onehot = (bt[None, :] == seg).astype(_f32)
    part = jnp.dot(onehot, h, preferred_element_type=_f32, precision=_HI)
    cpart = jnp.sum(onehot, axis=1, keepdims=True)

    @pl.when(i == 0)
    def _init():
        gsum_ref[...] = jnp.zeros_like(gsum_ref)
        cnt_ref[...] = jnp.zeros_like(cnt_ref)

    gsum_ref[...] += part
    cnt_ref[...] += cpart


def _b4_body(gsum_ref, cnt_ref, wl1_ref, bl1_ref, wl2_ref, bl2_ref, out_ref):
    g = gsum_ref[...] / jnp.maximum(cnt_ref[...], 1.0)
    a = jnp.dot(g, wl1_ref[...], preferred_element_type=_f32, precision=_HI)
    a = jnp.maximum(a + bl1_ref[...], 0.0)
    o = jnp.dot(a, wl2_ref[...], preferred_element_type=_f32, precision=_HI)
    o = o + bl2_ref[...]
    out_ref[...] = 1.0 / (1.0 + jnp.exp(-o))


def _b1_call(x, W1, hist):
    return pl.pallas_call(
        _b1_body,
        grid=(G,),
        in_specs=[
            pl.BlockSpec((R, D), lambda i: (i, 0)),
            pl.BlockSpec((D, D), lambda i: (0, 0)),
            pl.BlockSpec((NC, R, 16), lambda i: (0, i, 0)),
        ],
        out_specs=pl.BlockSpec((R, D), lambda i: (i, 0)),
        out_shape=jax.ShapeDtypeStruct((N, D), _f32),
    )(x, W1, hist)


def _b2_call(acc, hs, b, W, hist):
    return pl.pallas_call(
        _b2_body,
        grid=(G,),
        in_specs=[
            pl.BlockSpec((NC, R, D), lambda i: (0, i, 0)),
            pl.BlockSpec((R, D), lambda i: (i, 0)),
            pl.BlockSpec((1, D), lambda i: (0, 0)),
            pl.BlockSpec((D, D), lambda i: (0, 0)),
            pl.BlockSpec((NC, R, 16), lambda i: (0, i, 0)),
        ],
        out_specs=pl.BlockSpec((R, D), lambda i: (i, 0)),
        out_shape=jax.ShapeDtypeStruct((N, D), _f32),
    )(acc, hs, b, W, hist)


def _b3_call(acc, hs, b, hist, batch3):
    return pl.pallas_call(
        _b3_body,
        grid=(G,),
        in_specs=[
            pl.BlockSpec((NC, R, D), lambda i: (0, i, 0)),
            pl.BlockSpec((R, D), lambda i: (i, 0)),
            pl.BlockSpec((1, D), lambda i: (0, 0)),
            pl.BlockSpec((NC, R, 16), lambda i: (0, i, 0)),
            pl.BlockSpec((1, 1, R), lambda i: (i, 0, 0)),
        ],
        out_specs=[
            pl.BlockSpec((NG, D), lambda i: (0, 0)),
            pl.BlockSpec((NG, 1), lambda i: (0, 0)),
        ],
        out_shape=[
            jax.ShapeDtypeStruct((NG, D), _f32),
            jax.ShapeDtypeStruct((NG, 1), _f32),
        ],
    )(acc, hs, b, hist, batch3)


def _b4_call(gsum, cnt, Wl1, bl1, Wl2, bl2):
    return pl.pallas_call(
        _b4_body,
        out_shape=jax.ShapeDtypeStruct((NG, 5), _f32),
    )(gsum, cnt, Wl1, bl1, Wl2, bl2)


def kernel(x, edge_index, batch, W1, b1, W2, b2, Wl1, bl1, Wl2, bl2):
    src4 = edge_index[0].reshape(NW, HALVES, HALF, K)
    dst4 = edge_index[1].reshape(NW, HALVES, HALF, K)
    zrows = jnp.zeros((RPT, D), _f32)
    hist = _hist_kernel()(dst4).reshape(NC, N, 16)
    hs1 = _b1_call(x, W1, hist)
    acc1 = _scatter_kernel()(hs1, src4, dst4, zrows).reshape(NC, N, D)
    hs2 = _b2_call(acc1, hs1, b1.reshape(1, D), W2, hist)
    acc2 = _scatter_kernel()(hs2, src4, dst4, zrows).reshape(NC, N, D)
    gsum, cnt = _b3_call(acc2, hs2, b2.reshape(1, D), hist,
                         batch.reshape(G, 1, R))
    return _b4_call(gsum, cnt, Wl1, bl1.reshape(1, D // 2), Wl2,
                    bl2.reshape(1, 5))


# speedup vs baseline: 30.1401x; 1.0191x over previous
import functools

import jax
import jax.numpy as jnp
from jax import lax
from jax.experimental import pallas as pl
from jax.experimental.pallas import tpu as pltpu
from jax.experimental.pallas import tpu_sc as plsc

N = 10000
E = 320000
D = 128
NG = 16
NC = 2
NS = 16
NW = NC * NS
EPT = E // NW
K = 100
CHUNKS = EPT // K
HALVES = 2
HALF = CHUNKS // HALVES
WT = 10
RPT = N // WT
ZR = 40
R = 1000
G = N // R

_f32 = jnp.float32
_HI = lax.Precision.HIGHEST


def _hist_body(dst_hbm, out_hbm, dstb, ones_v, zb, deg_sh):
    cid = lax.axis_index("c")
    sid = lax.axis_index("s")
    wid = cid * NS + sid
    one = jnp.ones((16,), _f32)
    z = jnp.zeros((16,), _f32)

    def fill(r, _):
        ones_v[r, :] = one
        zb[r % ZR, :] = z
        return 0

    lax.fori_loop(0, K, fill, 0)

    @pl.when(sid < WT)
    def _zero():
        for k in range(RPT // ZR):
            pltpu.sync_copy(zb, deg_sh.at[pl.ds(sid * RPT + k * ZR, ZR)])

    plsc.subcore_barrier()

    def chunk(j, _):
        pltpu.sync_copy(ones_v, deg_sh.at[dstb.at[j]], add=True)
        return 0

    for h in range(HALVES):
        pltpu.sync_copy(dst_hbm.at[wid, h], dstb)
        lax.fori_loop(0, HALF, chunk, 0)

    plsc.subcore_barrier()

    @pl.when(sid < WT)
    def _wb():
        pltpu.sync_copy(deg_sh.at[pl.ds(sid * RPT, RPT)],
                        out_hbm.at[pl.ds(cid * N + sid * RPT, RPT)])


@functools.cache
def _hist_kernel():
    mesh = plsc.VectorSubcoreMesh(core_axis_name="c", subcore_axis_name="s")
    return pl.kernel(
        _hist_body,
        out_type=jax.ShapeDtypeStruct((NC * N, 16), _f32),
        mesh=mesh,
        scratch_types=[
            pltpu.VMEM((HALF, K), jnp.int32),
            pltpu.VMEM((K, 16), _f32),
            pltpu.VMEM((ZR, 16), _f32),
            pltpu.VMEM_SHARED((N, 16), _f32),
        ],
    )


def _scatter_body(hs_hbm, src_hbm, dst_hbm, zrows_hbm, out_hbm,
                  srcb, dstb, rows0, rows1, acc_sh, sem0, sem1):
    cid = lax.axis_index("c")
    sid = lax.axis_index("s")
    wid = cid * NS + sid

    @pl.when(sid < WT)
    def _zero():
        pltpu.sync_copy(zrows_hbm, acc_sh.at[pl.ds(sid * RPT, RPT)])

    plsc.subcore_barrier()

    def pair(t, _):
        j = 2 * t
        pltpu.async_copy(hs_hbm.at[srcb.at[j + 1]], rows1, sem1)
        pltpu.make_async_copy(hs_hbm.at[srcb.at[j]], rows0, sem0).wait()
        pltpu.sync_copy(rows0, acc_sh.at[dstb.at[j]], add=True)

        @pl.when(j + 2 < HALF)
        def _next():
            pltpu.async_copy(hs_hbm.at[srcb.at[j + 2]], rows0, sem0)

        pltpu.make_async_copy(hs_hbm.at[srcb.at[j + 1]], rows1, sem1).wait()
        pltpu.sync_copy(rows1, acc_sh.at[dstb.at[j + 1]], add=True)
        return 0

    for h in range(HALVES):
        pltpu.sync_copy(src_hbm.at[wid, h], srcb)
        pltpu.sync_copy(dst_hbm.at[wid, h], dstb)
        pltpu.async_copy(hs_hbm.at[srcb.at[0]], rows0, sem0)
        lax.fori_loop(0, HALF // 2, pair, 0)

    plsc.subcore_barrier()

    @pl.when(sid < WT)
    def _wb():
        pltpu.sync_copy(acc_sh.at[pl.ds(sid * RPT, RPT)],
                        out_hbm.at[pl.ds(cid * N + sid * RPT, RPT)])


@functools.cache
def _scatter_kernel():
    mesh = plsc.VectorSubcoreMesh(core_axis_name="c", subcore_axis_name="s")
    return pl.kernel(
        _scatter_body,
        out_type=jax.ShapeDtypeStruct((NC * N, D), _f32),
        mesh=mesh,
        scratch_types=[
            pltpu.VMEM((HALF, K), jnp.int32),
            pltpu.VMEM((HALF, K), jnp.int32),
            pltpu.VMEM((K, D), _f32),
            pltpu.VMEM((K, D), _f32),
            pltpu.VMEM_SHARED((N, D), _f32),
            pltpu.SemaphoreType.DMA,
            pltpu.SemaphoreType.DMA,
        ],
    )


def _dis_of(hist_ref):
    deg = 1.0 + hist_ref[0, :, 0] + hist_ref[1, :, 0]
    return lax.rsqrt(deg)


def _mm_body(x_ref, w_ref, out_ref):
    out_ref[...] = jnp.dot(x_ref[...], w_ref[...], preferred_element_type=_f32,
                           precision=_HI)


def _scale_body(hw_ref, hist_ref, out_ref):
    dis = _dis_of(hist_ref)
    out_ref[...] = hw_ref[...] * dis[:, None]


def _b2_body(acc_ref, hs_ref, b_ref, w_ref, hist_ref, out_ref):
    dis = _dis_of(hist_ref)
    h = dis[:, None] * (acc_ref[0] + acc_ref[1] + hs_ref[...]) + b_ref[...]
    h = jnp.maximum(h, 0.0)
    hw = jnp.dot(h, w_ref[...], preferred_element_type=_f32, precision=_HI)
    out_ref[...] = hw * dis[:, None]


def _b3_body(acc_ref, hs_ref, b_ref, hist_ref, batch_ref, wl1_ref, bl1_ref,
             wl2_ref, bl2_ref, out_ref, gsum_ref, cnt_ref):
    i = pl.program_id(0)
    dis = _dis_of(hist_ref)
    h = dis[:, None] * (acc_ref[0] + acc_ref[1] + hs_ref[...]) + b_ref[...]
    h = jnp.maximum(h, 0.0)
    bt = batch_ref[0, 0, :]
    seg = lax.broadcasted_iota(jnp.int32, (NG, R), 0)
    onehot = (bt[None, :] == seg).astype(_f32)
    part = jnp.dot(onehot, h, preferred_element_type=_f32, precision=_HI)
    cpart = jnp.sum(onehot, axis=1, keepdims=True)

    @pl.when(i == 0)
    def _init():
        gsum_ref[...] = jnp.zeros_like(gsum_ref)
        cnt_ref[...] = jnp.zeros_like(cnt_ref)

    gsum_ref[...] += part
    cnt_ref[...] += cpart

    @pl.when(i == G - 1)
    def _head():
        g = gsum_ref[...] / jnp.maximum(cnt_ref[...], 1.0)
        a = jnp.dot(g, wl1_ref[...], preferred_element_type=_f32,
                    precision=_HI)
        a = jnp.maximum(a + bl1_ref[...], 0.0)
        o = jnp.dot(a, wl2_ref[...], preferred_element_type=_f32,
                    precision=_HI)
        o = o + bl2_ref[...]
        out_ref[...] = 1.0 / (1.0 + jnp.exp(-o))


def _mm_call(x, W):
    return pl.pallas_call(
        _mm_body,
        grid=(G,),
        in_specs=[
            pl.BlockSpec((R, D), lambda i: (i, 0)),
            pl.BlockSpec((D, D), lambda i: (0, 0)),
        ],
        out_specs=pl.BlockSpec((R, D), lambda i: (i, 0)),
        out_shape=jax.ShapeDtypeStruct((N, D), _f32),
    )(x, W)


def _scale_call(hw, hist):
    return pl.pallas_call(
        _scale_body,
        grid=(G,),
        in_specs=[
            pl.BlockSpec((R, D), lambda i: (i, 0)),
            pl.BlockSpec((NC, R, 16), lambda i: (0, i, 0)),
        ],
        out_specs=pl.BlockSpec((R, D), lambda i: (i, 0)),
        out_shape=jax.ShapeDtypeStruct((N, D), _f32),
    )(hw, hist)


def _b2_call(acc, hs, b, W, hist):
    return pl.pallas_call(
        _b2_body,
        grid=(G,),
        in_specs=[
            pl.BlockSpec((NC, R, D), lambda i: (0, i, 0)),
            pl.BlockSpec((R, D), lambda i: (i, 0)),
            pl.BlockSpec((1, D), lambda i: (0, 0)),
            pl.BlockSpec((D, D), lambda i: (0, 0)),
            pl.BlockSpec((NC, R, 16), lambda i: (0, i, 0)),
        ],
        out_specs=pl.BlockSpec((R, D), lambda i: (i, 0)),
        out_shape=jax.ShapeDtypeStruct((N, D), _f32),
    )(acc, hs, b, W, hist)


def _b3_call(acc, hs, b, hist, batch3, Wl1, bl1, Wl2, bl2):
    return pl.pallas_call(
        _b3_body,
        grid=(G,),
        in_specs=[
            pl.BlockSpec((NC, R, D), lambda i: (0, i, 0)),
            pl.BlockSpec((R, D), lambda i: (i, 0)),
            pl.BlockSpec((1, D), lambda i: (0, 0)),
            pl.BlockSpec((NC, R, 16), lambda i: (0, i, 0)),
            pl.BlockSpec((1, 1, R), lambda i: (i, 0, 0)),
            pl.BlockSpec((D, D // 2), lambda i: (0, 0)),
            pl.BlockSpec((1, D // 2), lambda i: (0, 0)),
            pl.BlockSpec((D // 2, 5), lambda i: (0, 0)),
            pl.BlockSpec((1, 5), lambda i: (0, 0)),
        ],
        out_specs=pl.BlockSpec((NG, 5), lambda i: (0, 0)),
        out_shape=jax.ShapeDtypeStruct((NG, 5), _f32),
        scratch_shapes=[
            pltpu.VMEM((NG, D), _f32),
            pltpu.VMEM((NG, 1), _f32),
        ],
    )(acc, hs, b, hist, batch3, Wl1, bl1, Wl2, bl2)


def kernel(x, edge_index, batch, W1, b1, W2, b2, Wl1, bl1, Wl2, bl2):
    src4 = edge_index[0].reshape(NW, HALVES, HALF, K)
    dst4 = edge_index[1].reshape(NW, HALVES, HALF, K)
    zrows = jnp.zeros((RPT, D), _f32)
    hw1 = _mm_call(x, W1)
    hist = _hist_kernel()(dst4).reshape(NC, N, 16)
    hs1 = _scale_call(hw1, hist)
    acc1 = _scatter_kernel()(hs1, src4, dst4, zrows).reshape(NC, N, D)
    hs2 = _b2_call(acc1, hs1, b1.reshape(1, D), W2, hist)
    acc2 = _scatter_kernel()(hs2, src4, dst4, zrows).reshape(NC, N, D)
    return _b3_call(acc2, hs2, b2.reshape(1, D), hist,
                    batch.reshape(G, 1, R), Wl1, bl1.reshape(1, D // 2),
                    Wl2, bl2.reshape(1, 5))
